# baseline (speedup 1.0000x reference)
import functools

import jax
import jax.numpy as jnp
from jax import lax
from jax.experimental import pallas as pl
from jax.experimental.pallas import tpu as pltpu
from jax.experimental.pallas import tpu_sc as plsc

VOCAB = 1000000
EMB = 64
BATCH = 4096
SEQ = 200

N = BATCH * SEQ
NW = 32
PER_W = N // NW
CHUNK = 128
NCHUNK = PER_W // CHUNK

_mesh = plsc.VectorSubcoreMesh(core_axis_name="c", subcore_axis_name="s")


@functools.partial(
    pl.kernel,
    mesh=_mesh,
    out_type=jax.ShapeDtypeStruct((N, EMB), jnp.float32),
    scratch_types=[
        pltpu.VMEM((NCHUNK, CHUNK), jnp.int32),
        pltpu.VMEM((CHUNK, EMB), jnp.float32),
        pltpu.SemaphoreType.DMA,
    ],
    compiler_params=pltpu.CompilerParams(use_tc_tiling_on_sc=False),
)
def _gather_kernel(idx_hbm, table_hbm, out_hbm, idx_v, rows_v, sem):
    wid = lax.axis_index("s") * 2 + lax.axis_index("c")
    base = wid * PER_W
    pltpu.sync_copy(idx_hbm.at[wid], idx_v)

    def body(j, carry):
        pltpu.async_copy(table_hbm.at[idx_v.at[j]], rows_v, sem).wait()
        pltpu.sync_copy(rows_v, out_hbm.at[pl.ds(base + j * CHUNK, CHUNK)])
        return carry

    lax.fori_loop(0, NCHUNK, body, 0, unroll=False)


def kernel(x, table):
    idx = x.reshape(NW, NCHUNK, CHUNK).astype(jnp.int32)
    out = _gather_kernel(idx, table)
    return out.reshape(BATCH, SEQ, EMB)


# speedup vs baseline: 1.1172x; 1.1172x over previous
import functools

import jax
import jax.numpy as jnp
from jax import lax
from jax.experimental import pallas as pl
from jax.experimental.pallas import tpu as pltpu
from jax.experimental.pallas import tpu_sc as plsc

VOCAB = 1000000
EMB = 64
BATCH = 4096
SEQ = 200

N = BATCH * SEQ
NW = 32
PER_W = N // NW
CHUNK = 128
NCHUNK = PER_W // CHUNK
K = 4
BLOCK = K * CHUNK
NBLK = NCHUNK // K
PAIRS = NBLK // 2

_mesh = plsc.VectorSubcoreMesh(core_axis_name="c", subcore_axis_name="s")


@functools.partial(
    pl.kernel,
    mesh=_mesh,
    out_type=jax.ShapeDtypeStruct((N, EMB), jnp.float32),
    scratch_types=[
        pltpu.VMEM((NCHUNK, CHUNK), jnp.int32),
        pltpu.VMEM((BLOCK, EMB), jnp.float32),
        pltpu.VMEM((BLOCK, EMB), jnp.float32),
        pltpu.SemaphoreType.DMA,
        pltpu.SemaphoreType.DMA,
        pltpu.SemaphoreType.DMA,
        pltpu.SemaphoreType.DMA,
    ],
    compiler_params=pltpu.CompilerParams(use_tc_tiling_on_sc=False),
)
def _gather_kernel(idx_hbm, table_hbm, out_hbm, idx_v, rows_a, rows_b,
                   gsem_a, gsem_b, wsem_a, wsem_b):
    wid = lax.axis_index("s") * 2 + lax.axis_index("c")
    base = wid * PER_W
    pltpu.sync_copy(idx_hbm.at[wid], idx_v)

    def fire(blk, rows, gsem):
        for t in range(K):
            pltpu.async_copy(table_hbm.at[idx_v.at[blk * K + t]],
                             rows.at[pl.ds(t * CHUNK, CHUNK)], gsem)

    def drain_gather(rows, gsem):
        pltpu.make_async_copy(table_hbm.at[pl.ds(0, BLOCK)], rows, gsem).wait()

    def start_write(blk, rows, wsem):
        pltpu.async_copy(rows, out_hbm.at[pl.ds(base + blk * BLOCK, BLOCK)],
                         wsem)

    def wait_write(rows, wsem):
        pltpu.make_async_copy(rows, out_hbm.at[pl.ds(0, BLOCK)], wsem).wait()

    def pair(p, carry):
        blk0 = 2 * p

        @pl.when(p > 0)
        def _():
            wait_write(rows_a, wsem_a)

        fire(blk0, rows_a, gsem_a)

        @pl.when(p > 0)
        def _():
            drain_gather(rows_b, gsem_b)
            start_write(blk0 - 1, rows_b, wsem_b)

        @pl.when(p > 0)
        def _():
            wait_write(rows_b, wsem_b)

        fire(blk0 + 1, rows_b, gsem_b)
        drain_gather(rows_a, gsem_a)
        start_write(blk0, rows_a, wsem_a)
        return carry

    lax.fori_loop(0, PAIRS, pair, 0, unroll=False)

    drain_gather(rows_b, gsem_b)
    start_write(NBLK - 1, rows_b, wsem_b)
    wait_write(rows_a, wsem_a)
    wait_write(rows_b, wsem_b)


def kernel(x, table):
    idx = x.reshape(NW, NCHUNK, CHUNK).astype(jnp.int32)
    out = _gather_kernel(idx, table)
    return out.reshape(BATCH, SEQ, EMB)
